# Initial kernel scaffold; baseline (speedup 1.0000x reference)
import jax
import jax.numpy as jnp
from jax.experimental import pallas as pl


def kernel(x, W):
    raise NotImplementedError("write your pallas kernel here")


# speedup vs baseline: 1.9711x; 1.9711x over previous
import functools

import jax
import jax.numpy as jnp
from jax.experimental import pallas as pl

_HIDDEN = 768
_NUM_EXPERTS = 64
_TOP_K = 2
_BLK = 1024


def _router_kernel(x_ref, w_ref, logits_ref, weights_ref, idx_ref):
    x_blk = x_ref[...]
    w = w_ref[...]
    logits = jax.lax.dot_general(
        x_blk, w,
        dimension_numbers=(((1,), (1,)), ((), ())),
        preferred_element_type=jnp.float32,
    )
    logits_ref[...] = logits

    lane = jax.lax.broadcasted_iota(jnp.int32, logits.shape, 1)
    m1 = jnp.max(logits, axis=1, keepdims=True)
    i1 = jnp.min(jnp.where(logits == m1, lane, _NUM_EXPERTS), axis=1,
                 keepdims=True)
    masked = jnp.where(lane == i1, -jnp.inf, logits)
    m2 = jnp.max(masked, axis=1, keepdims=True)
    i2 = jnp.min(jnp.where(masked == m2, lane, _NUM_EXPERTS), axis=1,
                 keepdims=True)

    e2 = jnp.exp(m2 - m1)
    w1 = 1.0 / (1.0 + e2)
    w2 = 1.0 - w1

    weights_ref[...] = jnp.concatenate([w1, w2], axis=1)
    idx_ref[...] = jnp.concatenate([i1, i2], axis=1)


@jax.jit
def kernel(x, W):
    batch, seq_len, hidden = x.shape
    n_rows = batch * seq_len
    x_flat = x.reshape(n_rows, hidden)

    grid = (n_rows // _BLK,)
    logits, weights, idx = pl.pallas_call(
        _router_kernel,
        grid=grid,
        in_specs=[
            pl.BlockSpec((_BLK, hidden), lambda i: (i, 0)),
            pl.BlockSpec((_NUM_EXPERTS, hidden), lambda i: (0, 0)),
        ],
        out_specs=[
            pl.BlockSpec((_BLK, _NUM_EXPERTS), lambda i: (i, 0)),
            pl.BlockSpec((_BLK, _TOP_K), lambda i: (i, 0)),
            pl.BlockSpec((_BLK, _TOP_K), lambda i: (i, 0)),
        ],
        out_shape=[
            jax.ShapeDtypeStruct((n_rows, _NUM_EXPERTS), jnp.float32),
            jax.ShapeDtypeStruct((n_rows, _TOP_K), jnp.float32),
            jax.ShapeDtypeStruct((n_rows, _TOP_K), jnp.int32),
        ],
    )(x_flat, W)

    return (logits.reshape(batch, seq_len, _NUM_EXPERTS), weights, idx)


# speedup vs baseline: 2.0633x; 1.0468x over previous
import functools

import jax
import jax.numpy as jnp
from jax.experimental import pallas as pl

_HIDDEN = 768
_NUM_EXPERTS = 64
_TOP_K = 2
_BLK = 1024


def _router_kernel(x_ref, w_ref, logits_ref, weights_ref, idx_ref):
    x_blk = x_ref[...]
    w = w_ref[...]
    logits = jax.lax.dot_general(
        x_blk, w,
        dimension_numbers=(((1,), (1,)), ((), ())),
        preferred_element_type=jnp.float32,
    )
    logits_ref[...] = logits

    lane = jax.lax.broadcasted_iota(jnp.int32, logits.shape, 1).astype(
        jnp.float32)
    m1 = jnp.max(logits, axis=1, keepdims=True)
    i1 = jnp.min(jnp.where(logits == m1, lane, _NUM_EXPERTS), axis=1,
                 keepdims=True)
    masked = jnp.where(lane == i1, -jnp.inf, logits)
    m2 = jnp.max(masked, axis=1, keepdims=True)
    i2 = jnp.min(jnp.where(masked == m2, lane, _NUM_EXPERTS), axis=1,
                 keepdims=True)

    e2 = jnp.exp(m2 - m1)
    w1 = 1.0 / (1.0 + e2)
    w2 = 1.0 - w1

    weights_ref[...] = jnp.concatenate([w1, w2], axis=1)
    idx_ref[...] = jnp.concatenate([i1, i2], axis=1).astype(jnp.int32)


@jax.jit
def kernel(x, W):
    batch, seq_len, hidden = x.shape
    n_rows = batch * seq_len
    x_flat = x.reshape(n_rows, hidden)

    grid = (n_rows // _BLK,)
    logits, weights, idx = pl.pallas_call(
        _router_kernel,
        grid=grid,
        in_specs=[
            pl.BlockSpec((_BLK, hidden), lambda i: (i, 0)),
            pl.BlockSpec((_NUM_EXPERTS, hidden), lambda i: (0, 0)),
        ],
        out_specs=[
            pl.BlockSpec((_BLK, _NUM_EXPERTS), lambda i: (i, 0)),
            pl.BlockSpec((_BLK, _TOP_K), lambda i: (i, 0)),
            pl.BlockSpec((_BLK, _TOP_K), lambda i: (i, 0)),
        ],
        out_shape=[
            jax.ShapeDtypeStruct((n_rows, _NUM_EXPERTS), jnp.float32),
            jax.ShapeDtypeStruct((n_rows, _TOP_K), jnp.float32),
            jax.ShapeDtypeStruct((n_rows, _TOP_K), jnp.int32),
        ],
    )(x_flat, W)

    return (logits.reshape(batch, seq_len, _NUM_EXPERTS), weights, idx)


# speedup vs baseline: 2.2400x; 1.0857x over previous
import functools

import jax
import jax.numpy as jnp
from jax.experimental import pallas as pl

_HIDDEN = 768
_NUM_EXPERTS = 64
_TOP_K = 2
_BLK = 2048


def _router_kernel(x_ref, w_ref, logits_ref, weights_ref, idx_ref):
    x_blk = x_ref[...]
    w = w_ref[...]
    logits = jax.lax.dot_general(
        x_blk, w,
        dimension_numbers=(((1,), (1,)), ((), ())),
        preferred_element_type=jnp.float32,
    )
    logits_ref[...] = logits

    lane = jax.lax.broadcasted_iota(jnp.int32, logits.shape, 1).astype(
        jnp.float32)
    m1 = jnp.max(logits, axis=1, keepdims=True)
    i1 = jnp.min(jnp.where(logits == m1, lane, _NUM_EXPERTS), axis=1,
                 keepdims=True)
    masked = jnp.where(lane == i1, -jnp.inf, logits)
    m2 = jnp.max(masked, axis=1, keepdims=True)
    i2 = jnp.min(jnp.where(masked == m2, lane, _NUM_EXPERTS), axis=1,
                 keepdims=True)

    e2 = jnp.exp(m2 - m1)
    w1 = 1.0 / (1.0 + e2)
    w2 = 1.0 - w1

    weights_ref[...] = jnp.concatenate([w1, w2], axis=1)
    idx_ref[...] = jnp.concatenate([i1, i2], axis=1).astype(jnp.int32)


@jax.jit
def kernel(x, W):
    batch, seq_len, hidden = x.shape
    n_rows = batch * seq_len
    x_flat = x.reshape(n_rows, hidden)

    grid = (n_rows // _BLK,)
    logits, weights, idx = pl.pallas_call(
        _router_kernel,
        grid=grid,
        in_specs=[
            pl.BlockSpec((_BLK, hidden), lambda i: (i, 0)),
            pl.BlockSpec((_NUM_EXPERTS, hidden), lambda i: (0, 0)),
        ],
        out_specs=[
            pl.BlockSpec((_BLK, _NUM_EXPERTS), lambda i: (i, 0)),
            pl.BlockSpec((_BLK, _TOP_K), lambda i: (i, 0)),
            pl.BlockSpec((_BLK, _TOP_K), lambda i: (i, 0)),
        ],
        out_shape=[
            jax.ShapeDtypeStruct((n_rows, _NUM_EXPERTS), jnp.float32),
            jax.ShapeDtypeStruct((n_rows, _TOP_K), jnp.float32),
            jax.ShapeDtypeStruct((n_rows, _TOP_K), jnp.int32),
        ],
    )(x_flat, W)

    return (logits.reshape(batch, seq_len, _NUM_EXPERTS), weights, idx)


# speedup vs baseline: 2.3333x; 1.0417x over previous
import functools

import jax
import jax.numpy as jnp
from jax.experimental import pallas as pl

_HIDDEN = 768
_NUM_EXPERTS = 64
_TOP_K = 2
_BLK = 4096


def _router_kernel(x_ref, w_ref, logits_ref, weights_ref, idx_ref):
    x_blk = x_ref[...]
    w = w_ref[...]
    logits = jax.lax.dot_general(
        x_blk, w,
        dimension_numbers=(((1,), (1,)), ((), ())),
        preferred_element_type=jnp.float32,
    )
    logits_ref[...] = logits

    lane = jax.lax.broadcasted_iota(jnp.int32, logits.shape, 1).astype(
        jnp.float32)
    m1 = jnp.max(logits, axis=1, keepdims=True)
    i1 = jnp.min(jnp.where(logits == m1, lane, _NUM_EXPERTS), axis=1,
                 keepdims=True)
    masked = jnp.where(lane == i1, -jnp.inf, logits)
    m2 = jnp.max(masked, axis=1, keepdims=True)
    i2 = jnp.min(jnp.where(masked == m2, lane, _NUM_EXPERTS), axis=1,
                 keepdims=True)

    e2 = jnp.exp(m2 - m1)
    w1 = 1.0 / (1.0 + e2)
    w2 = 1.0 - w1

    weights_ref[...] = jnp.concatenate([w1, w2], axis=1)
    idx_ref[...] = jnp.concatenate([i1, i2], axis=1).astype(jnp.int32)


@jax.jit
def kernel(x, W):
    batch, seq_len, hidden = x.shape
    n_rows = batch * seq_len
    x_flat = x.reshape(n_rows, hidden)

    grid = (n_rows // _BLK,)
    logits, weights, idx = pl.pallas_call(
        _router_kernel,
        grid=grid,
        in_specs=[
            pl.BlockSpec((_BLK, hidden), lambda i: (i, 0)),
            pl.BlockSpec((_NUM_EXPERTS, hidden), lambda i: (0, 0)),
        ],
        out_specs=[
            pl.BlockSpec((_BLK, _NUM_EXPERTS), lambda i: (i, 0)),
            pl.BlockSpec((_BLK, _TOP_K), lambda i: (i, 0)),
            pl.BlockSpec((_BLK, _TOP_K), lambda i: (i, 0)),
        ],
        out_shape=[
            jax.ShapeDtypeStruct((n_rows, _NUM_EXPERTS), jnp.float32),
            jax.ShapeDtypeStruct((n_rows, _TOP_K), jnp.float32),
            jax.ShapeDtypeStruct((n_rows, _TOP_K), jnp.int32),
        ],
    )(x_flat, W)

    return (logits.reshape(batch, seq_len, _NUM_EXPERTS), weights, idx)
